# baseline (speedup 1.0000x reference)
import functools

import numpy as np
import jax
import jax.numpy as jnp
from jax import lax
from jax.experimental import pallas as pl
from jax.experimental.pallas import tpu as pltpu
from jax.experimental.pallas import tpu_sc as plsc

_LANES = 16
_HALF = 112

_PERM_CACHE = {}


def _perm(N):
    if N not in _PERM_CACHE:
        cpu = jax.local_devices(backend="cpu")[0]
        with jax.default_device(cpu), jax.ensure_compile_time_eval():
            r = np.asarray(jax.random.permutation(jax.random.key(42), N))
        _PERM_CACHE[N] = r.astype(np.int32)
    return _PERM_CACHE[N]


def _copy_rows(src_ref, dst_ref, n_rows, width):
    def body(r, carry):
        for c in range(width // _LANES):
            dst_ref[r, pl.ds(c * _LANES, _LANES)] = src_ref[
                r, pl.ds(c * _LANES, _LANES)
            ]
        return carry

    lax.fori_loop(0, n_rows, body, 0)


@functools.lru_cache(maxsize=None)
def _make_repack(H, W, C):
    info = plsc.get_sparse_core_info()
    NW = info.num_cores * info.num_subcores
    NC = info.num_cores
    planes_per_w = H // NW

    mesh = plsc.VectorSubcoreMesh(core_axis_name="c", subcore_axis_name="s")

    @functools.partial(
        pl.kernel,
        mesh=mesh,
        out_type=jax.ShapeDtypeStruct((H * W, 128), jnp.float32),
        scratch_types=[
            pltpu.VMEM((W, C), jnp.float32),
            pltpu.VMEM((W, 128), jnp.float32),
        ],
    )
    def repack(x_hbm, xp_hbm, buf96_v, buf128_v):
        wid = lax.axis_index("s") * NC + lax.axis_index("c")

        def body(p, carry):
            h = wid * planes_per_w + p
            pltpu.sync_copy(x_hbm.at[0, h], buf96_v)
            _copy_rows(buf96_v, buf128_v, W, C)
            pltpu.sync_copy(buf128_v, xp_hbm.at[pl.ds(h * W, W)])
            return carry

        lax.fori_loop(0, planes_per_w, body, 0)

    return repack


@functools.lru_cache(maxsize=None)
def _make_gather(H, W, C):
    info = plsc.get_sparse_core_info()
    NW = info.num_cores * info.num_subcores
    NC = info.num_cores
    planes_per_w = H // NW
    n_half = W * planes_per_w // _HALF

    mesh = plsc.VectorSubcoreMesh(core_axis_name="c", subcore_axis_name="s")

    @functools.partial(
        pl.kernel,
        mesh=mesh,
        out_type=jax.ShapeDtypeStruct((1, H, W, C), jnp.float32),
        scratch_types=[
            pltpu.VMEM((n_half, _HALF), jnp.int32),
            pltpu.VMEM((W, 128), jnp.float32),
            pltpu.VMEM((W, C), jnp.float32),
            pltpu.SemaphoreType.DMA,
        ],
    )
    def gather(xp_hbm, idx_hbm, out_hbm, idx_v, buf128_v, buf96_v, sem):
        wid = lax.axis_index("s") * NC + lax.axis_index("c")
        pltpu.sync_copy(idx_hbm.at[wid], idx_v)

        def body(p, carry):
            h = wid * planes_per_w + p
            cp0 = pltpu.async_copy(
                xp_hbm.at[idx_v.at[2 * p]], buf128_v.at[pl.ds(0, _HALF)], sem
            )
            cp1 = pltpu.async_copy(
                xp_hbm.at[idx_v.at[2 * p + 1]], buf128_v.at[pl.ds(_HALF, _HALF)], sem
            )
            cp0.wait()
            cp1.wait()
            _copy_rows(buf128_v, buf96_v, W, C)
            pltpu.sync_copy(buf96_v, out_hbm.at[0, h])
            return carry

        lax.fori_loop(0, planes_per_w, body, 0)

    return gather


def kernel(x):
    B, H, W, C = x.shape
    N = H * W
    idx = jnp.asarray(_perm(N).reshape(32, -1, _HALF))
    repack = _make_repack(H, W, C)
    gather = _make_gather(H, W, C)
    out = jnp.zeros((B, H, W, C), jnp.float32)
    for b in range(B):
        xb = lax.slice(x, (b, 0, 0, 0), (b + 1, H, W, C))
        xpb = repack(xb)
        piece = gather(xpb, idx)
        out = lax.dynamic_update_slice(out, piece, (b, 0, 0, 0))
    return out


# speedup vs baseline: 1.5131x; 1.5131x over previous
import functools

import numpy as np
import jax
import jax.numpy as jnp
from jax import lax
from jax.experimental import pallas as pl
from jax.experimental.pallas import tpu as pltpu
from jax.experimental.pallas import tpu_sc as plsc

_LANES = 16
_HALF = 112

_PERM_CACHE = {}


def _full_index(B, N):
    key = (B, N)
    if key not in _PERM_CACHE:
        cpu = jax.local_devices(backend="cpu")[0]
        with jax.default_device(cpu), jax.ensure_compile_time_eval():
            r = np.asarray(jax.random.permutation(jax.random.key(42), N))
        idx = (np.arange(B, dtype=np.int64)[:, None] * N + r[None, :]).reshape(-1)
        _PERM_CACHE[key] = idx.astype(np.int32)
    return _PERM_CACHE[key]


def _copy_rows(src_ref, dst_ref, k, n_rows, width):
    def body(r, carry):
        for c in range(width // _LANES):
            dst_ref[k, r, pl.ds(c * _LANES, _LANES)] = src_ref[
                k, r, pl.ds(c * _LANES, _LANES)
            ]
        return carry

    lax.fori_loop(0, n_rows, body, 0)


@functools.lru_cache(maxsize=None)
def _make_shuffle(B, H, W, C):
    info = plsc.get_sparse_core_info()
    NW = info.num_cores * info.num_subcores
    R = B * H * W
    slabs = NW // B
    npl = H // slabs
    assert W == 2 * _HALF

    mesh = plsc.VectorSubcoreMesh(core_axis_name="c", subcore_axis_name="s")

    @functools.partial(
        pl.kernel,
        mesh=mesh,
        out_type=(
            jax.ShapeDtypeStruct((B, H, W, C), jnp.float32),
            jax.ShapeDtypeStruct((R, 128), jnp.float32),
        ),
        scratch_types=[
            pltpu.VMEM((2, W, C), jnp.float32),
            pltpu.VMEM((2, W, 128), jnp.float32),
            pltpu.VMEM((4, _HALF), jnp.int32),
        ]
        + [pltpu.SemaphoreType.DMA] * 8,
    )
    def shuffle(
        x_hbm, idx_hbm, out_hbm, xp_hbm, b96, b128, idxb,
        sin0, sin1, sout0, sout1, sg0, sg1, si0, si1,
    ):
        sin = (sin0, sin1)
        sout = (sout0, sout1)
        sg = (sg0, sg1)
        si = (si0, si1)
        cid = lax.axis_index("c")
        sid = lax.axis_index("s")
        b = cid * (B // 2) + sid // slabs
        h0 = (sid % slabs) * npl
        kidx = b * slabs + sid % slabs

        def start_in(k, p):
            pltpu.async_copy(x_hbm.at[b, h0 + p], b96.at[k], sin[k])

        def wait_in(k):
            pltpu.make_async_copy(x_hbm.at[b, h0], b96.at[k], sin[k]).wait()

        def start_out(k, p):
            pltpu.async_copy(
                b128.at[k], xp_hbm.at[pl.ds((b * H + h0 + p) * W, W)], sout[k]
            )

        def wait_out(k):
            pltpu.make_async_copy(
                b128.at[k], xp_hbm.at[pl.ds(b * H * W, W)], sout[k]
            ).wait()

        start_in(0, 0)
        start_in(1, 1)

        def repack(g, carry):
            for k in range(2):
                p = 2 * g + k
                wait_in(k)

                @pl.when(p >= 2)
                def _():
                    wait_out(k)

                _copy_rows(b96, b128, k, W, C)
                start_out(k, p)

                @pl.when(p + 2 < npl)
                def _():
                    start_in(k, p + 2)

            return carry

        lax.fori_loop(0, npl // 2, repack, 0)
        wait_out(0)
        wait_out(1)
        plsc.subcore_barrier()

        def start_idx(k, p):
            pltpu.async_copy(idx_hbm.at[kidx, p], idxb.at[pl.ds(2 * k, 2)], si[k])

        def wait_idx(k):
            pltpu.make_async_copy(
                idx_hbm.at[kidx, 0], idxb.at[pl.ds(2 * k, 2)], si[k]
            ).wait()

        def start_g(k):
            pltpu.async_copy(
                xp_hbm.at[idxb.at[2 * k]], b128.at[k].at[pl.ds(0, _HALF)], sg[k]
            )
            pltpu.async_copy(
                xp_hbm.at[idxb.at[2 * k + 1]], b128.at[k].at[pl.ds(_HALF, _HALF)], sg[k]
            )

        def wait_g(k):
            for _ in range(2):
                pltpu.make_async_copy(
                    xp_hbm.at[idxb.at[2 * k]], b128.at[k].at[pl.ds(0, _HALF)], sg[k]
                ).wait()

        def start_o96(k, p):
            pltpu.async_copy(b96.at[k], out_hbm.at[b, h0 + p], sout[k])

        def wait_o96(k):
            pltpu.make_async_copy(b96.at[k], out_hbm.at[b, h0], sout[k]).wait()

        start_idx(0, 0)
        start_idx(1, 1)
        wait_idx(0)
        start_g(0)

        def gather(g, carry):
            for k in range(2):
                p = 2 * g + k

                @pl.when(p + 1 < npl)
                def _():
                    wait_idx(1 - k)
                    start_g(1 - k)

                wait_g(k)

                @pl.when(p >= 2)
                def _():
                    wait_o96(k)

                _copy_rows(b128, b96, k, W, C)
                start_o96(k, p)

                @pl.when(p + 2 < npl)
                def _():
                    start_idx(k, p + 2)

            return carry

        lax.fori_loop(0, npl // 2, gather, 0)
        wait_o96(0)
        wait_o96(1)

    return shuffle


def kernel(x):
    B, H, W, C = x.shape
    N = H * W
    idx = jnp.asarray(_full_index(B, N).reshape(32, -1, 2, _HALF))
    out, _ = _make_shuffle(B, H, W, C)(x, idx)
    return out
